# baseline (speedup 1.0000x reference)
import dataclasses
import functools

import jax
import jax.numpy as jnp
from jax import lax
from jax.experimental import pallas as pl
from jax.experimental.pallas import tpu as pltpu
from jax.experimental.pallas import tpu_sc as plsc

N = 10000
NP = 10240
F = 128
E = 320000
NC, NS = 2, 16
NT = NC * NS
CHUNK = 128
CPT = 80
EPT = CPT * CHUNK
EP = NT * EPT
RPS = NP // NS

_mesh = plsc.VectorSubcoreMesh(core_axis_name="c", subcore_axis_name="s")

_sc_params = pltpu.CompilerParams()
if "needs_layout_passes" in pltpu.CompilerParams.__dataclass_fields__:
    _sc_params = dataclasses.replace(_sc_params, needs_layout_passes=False)


@jax.jit
def _sc_degree(dst3):
    @functools.partial(
        pl.kernel,
        out_type=jax.ShapeDtypeStruct((NT, NP), jnp.float32),
        mesh=_mesh,
        compiler_params=_sc_params,
        scratch_types=[
            pltpu.VMEM((CPT, CHUNK), jnp.int32),
            pltpu.VMEM((NP,), jnp.float32),
        ],
    )
    def k(dst_hbm, out_hbm, dstv, hist):
        c = lax.axis_index("c")
        s = lax.axis_index("s")
        wid = s * NC + c

        @pl.loop(0, NP, step=16)
        def _zero(i):
            hist[pl.ds(i, 16)] = jnp.zeros((16,), jnp.float32)

        pltpu.sync_copy(dst_hbm.at[wid], dstv)

        @pl.loop(0, CPT)
        def _chunk(j):
            @pl.loop(0, CHUNK, step=16)
            def _vec(q):
                idx = dstv[j, pl.ds(q, 16)]
                plsc.addupdate_scatter(hist, [idx], jnp.ones((16,), jnp.float32))

        pltpu.sync_copy(hist, out_hbm.at[wid])

    return k(dst3)


@jax.jit
def _sc_aggregate(hp, src3, dst3):
    @functools.partial(
        pl.kernel,
        out_type=jax.ShapeDtypeStruct((NC, NP, F), jnp.float32),
        mesh=_mesh,
        scratch_types=[
            pltpu.VMEM((CPT, CHUNK), jnp.int32),
            pltpu.VMEM((8, CHUNK), jnp.int32),
            pltpu.VMEM((CHUNK, F), jnp.float32),
            pltpu.VMEM((CHUNK, F), jnp.float32),
            pltpu.VMEM_SHARED((NP, F), jnp.float32),
            pltpu.SemaphoreType.DMA,
            pltpu.SemaphoreType.DMA,
        ],
    )
    def k(hp_hbm, src_hbm, dst_hbm, out_hbm, srcv, dstv, bufa, bufb, acc,
          gs, ss):
        c = lax.axis_index("c")
        s = lax.axis_index("s")
        wid = s * NC + c
        base = s * RPS

        @pl.loop(0, CHUNK)
        def _zrow(i):
            @pl.loop(0, F, step=16)
            def _zcol(j):
                bufa[i, pl.ds(j, 16)] = jnp.zeros((16,), jnp.float32)

        @pl.loop(0, RPS, step=CHUNK)
        def _zacc(r):
            pltpu.sync_copy(bufa, acc.at[pl.ds(base + r, CHUNK)])

        plsc.subcore_barrier()

        pltpu.sync_copy(src_hbm.at[wid], srcv)
        for p in range(CPT // 8):
            pltpu.sync_copy(dst_hbm.at[wid, pl.ds(p * 8, 8)], dstv)

            @pl.loop(0, 8, step=2)
            def _edge(j):
                jj = p * 8 + j
                d0 = pltpu.async_copy(hp_hbm.at[srcv.at[jj]], bufa, gs)
                d1 = pltpu.async_copy(hp_hbm.at[srcv.at[jj + 1]], bufb, gs)
                d0.wait()
                s0 = pltpu.async_copy(bufa, acc.at[dstv.at[j]], ss, add=True)
                d1.wait()
                s1 = pltpu.async_copy(bufb, acc.at[dstv.at[j + 1]], ss,
                                      add=True)
                s0.wait()
                s1.wait()

        plsc.subcore_barrier()
        pltpu.sync_copy(acc.at[pl.ds(base, RPS)], out_hbm.at[c, pl.ds(base, RPS)])

    return k(hp, src3, dst3)


_R = 1024


def _dot(a, b):
    return jax.lax.dot_general(a, b, (((1,), (0,)), ((), ())),
                               preferred_element_type=jnp.float32)


@jax.jit
def _tc_mm(xp, W1):
    def body(x_ref, w_ref, o_ref):
        o_ref[...] = _dot(x_ref[...], w_ref[...])

    return pl.pallas_call(
        body,
        grid=(NP // _R,),
        in_specs=[
            pl.BlockSpec((_R, F), lambda i: (i, 0)),
            pl.BlockSpec((F, F), lambda i: (0, 0)),
        ],
        out_specs=pl.BlockSpec((_R, F), lambda i: (i, 0)),
        out_shape=jax.ShapeDtypeStruct((NP, F), jnp.float32),
    )(xp, W1)


@jax.jit
def _tc_scale(g, dinv):
    def body(g_ref, d_ref, o_ref):
        o_ref[...] = d_ref[...] * g_ref[...]

    return pl.pallas_call(
        body,
        grid=(NP // _R,),
        in_specs=[
            pl.BlockSpec((_R, F), lambda i: (i, 0)),
            pl.BlockSpec((_R, 1), lambda i: (i, 0)),
        ],
        out_specs=pl.BlockSpec((_R, F), lambda i: (i, 0)),
        out_shape=jax.ShapeDtypeStruct((NP, F), jnp.float32),
    )(g, dinv)


@jax.jit
def _tc_layer(p, hp, dinv, sc, sh, Wn):
    def body(p0_ref, p1_ref, hp_ref, d_ref, sc_ref, sh_ref, w_ref, o_ref):
        z = p0_ref[0] + p1_ref[0] + hp_ref[...]
        y = jnp.maximum(d_ref[...] * z * sc_ref[...] + sh_ref[...], 0.0)
        o_ref[...] = d_ref[...] * _dot(y, w_ref[...])

    return pl.pallas_call(
        body,
        grid=(NP // _R,),
        in_specs=[
            pl.BlockSpec((1, _R, F), lambda i: (0, i, 0)),
            pl.BlockSpec((1, _R, F), lambda i: (1, i, 0)),
            pl.BlockSpec((_R, F), lambda i: (i, 0)),
            pl.BlockSpec((_R, 1), lambda i: (i, 0)),
            pl.BlockSpec((1, F), lambda i: (0, 0)),
            pl.BlockSpec((1, F), lambda i: (0, 0)),
            pl.BlockSpec((F, F), lambda i: (0, 0)),
        ],
        out_specs=pl.BlockSpec((_R, F), lambda i: (i, 0)),
        out_shape=jax.ShapeDtypeStruct((NP, F), jnp.float32),
    )(p, p, hp, dinv, sc, sh, Wn)


@jax.jit
def _tc_head(p, hp, dinv, sc, sh, M1, c1, M2, c2, M3, c3, M4, c4):
    def body(p0_ref, p1_ref, hp_ref, d_ref, sc_ref, sh_ref,
             m1, c1r, m2, c2r, m3, c3r, m4, c4r, o_ref):
        z = p0_ref[0] + p1_ref[0] + hp_ref[...]
        y = jnp.maximum(d_ref[...] * z * sc_ref[...] + sh_ref[...], 0.0)
        t = jnp.maximum(_dot(y, m1[...]) + c1r[...], 0.0)
        t = jnp.maximum(_dot(t, m2[...]) + c2r[...], 0.0)
        t = jnp.maximum(_dot(t, m3[...]) + c3r[...], 0.0)
        o_ref[...] = _dot(t, m4[...]) + c4r[...]

    return pl.pallas_call(
        body,
        grid=(NP // _R,),
        in_specs=[
            pl.BlockSpec((1, _R, F), lambda i: (0, i, 0)),
            pl.BlockSpec((1, _R, F), lambda i: (1, i, 0)),
            pl.BlockSpec((_R, F), lambda i: (i, 0)),
            pl.BlockSpec((_R, 1), lambda i: (i, 0)),
            pl.BlockSpec((1, F), lambda i: (0, 0)),
            pl.BlockSpec((1, F), lambda i: (0, 0)),
            pl.BlockSpec((F, 128), lambda i: (0, 0)),
            pl.BlockSpec((1, 128), lambda i: (0, 0)),
            pl.BlockSpec((128, 64), lambda i: (0, 0)),
            pl.BlockSpec((1, 64), lambda i: (0, 0)),
            pl.BlockSpec((64, 32), lambda i: (0, 0)),
            pl.BlockSpec((1, 32), lambda i: (0, 0)),
            pl.BlockSpec((32, 16), lambda i: (0, 0)),
            pl.BlockSpec((1, 16), lambda i: (0, 0)),
        ],
        out_specs=pl.BlockSpec((_R, 16), lambda i: (i, 0)),
        out_shape=jax.ShapeDtypeStruct((N, 16), jnp.float32),
    )(p, p, hp, dinv, sc, sh, M1, c1, M2, c2, M3, c3, M4, c4)


def kernel(x, W1, b1, g1, be1, rm1, rv1, W2, b2, g2, be2, rm2, rv2,
           W3, b3, g3, be3, rm3, rv3, lw1, lb1, lw2, lb2, lw3, lb3,
           lw4, lb4, edge_index):
    ei = edge_index.astype(jnp.int32)
    pad = N + jnp.arange(EP - E, dtype=jnp.int32) % (NP - N)
    src3 = jnp.concatenate([ei[0], pad]).reshape(NT, CPT, CHUNK)
    dst3 = jnp.concatenate([ei[1], pad]).reshape(NT, CPT, CHUNK)
    xp = jnp.pad(x, ((0, NP - N), (0, 0)))

    xw1 = _tc_mm(xp, W1)
    hist = _sc_degree(dst3)
    deg = jnp.sum(hist, axis=0) + 1.0
    dinv = jnp.where(jnp.arange(NP) < N, lax.rsqrt(deg), 0.0).reshape(NP, 1)

    def bn_consts(b, g, be, rm, rv):
        s = g * lax.rsqrt(rv + 1e-5)
        return s.reshape(1, F), ((b - rm) * s + be).reshape(1, F)

    s1, t1 = bn_consts(b1, g1, be1, rm1, rv1)
    s2, t2 = bn_consts(b2, g2, be2, rm2, rv2)
    s3, t3 = bn_consts(b3, g3, be3, rm3, rv3)

    hp = _tc_scale(xw1, dinv)
    p = _sc_aggregate(hp, src3, dst3)
    hp = _tc_layer(p, hp, dinv, s1, t1, W2)
    p = _sc_aggregate(hp, src3, dst3)
    hp = _tc_layer(p, hp, dinv, s2, t2, W3)
    p = _sc_aggregate(hp, src3, dst3)
    out = _tc_head(p, hp, dinv, s3, t3,
                   lw1.T, lb1.reshape(1, -1), lw2.T, lb2.reshape(1, -1),
                   lw3.T, lb3.reshape(1, -1), lw4.T, lb4.reshape(1, -1))
    return out


# speedup vs baseline: 1.0213x; 1.0213x over previous
import dataclasses
import functools

import jax
import jax.numpy as jnp
from jax import lax
from jax.experimental import pallas as pl
from jax.experimental.pallas import tpu as pltpu
from jax.experimental.pallas import tpu_sc as plsc

N = 10000
NP = 10240
F = 128
E = 320000
NC, NS = 2, 16
NT = NC * NS
CHUNK = 128
CPT = 80
EPT = CPT * CHUNK
EP = NT * EPT
RPS = NP // NS

_mesh = plsc.VectorSubcoreMesh(core_axis_name="c", subcore_axis_name="s")

_sc_params = pltpu.CompilerParams()
if "needs_layout_passes" in pltpu.CompilerParams.__dataclass_fields__:
    _sc_params = dataclasses.replace(_sc_params, needs_layout_passes=False)


@jax.jit
def _sc_degree(dst3):
    @functools.partial(
        pl.kernel,
        out_type=jax.ShapeDtypeStruct((NT, NP), jnp.float32),
        mesh=_mesh,
        compiler_params=_sc_params,
        scratch_types=[
            pltpu.VMEM((CPT, CHUNK), jnp.int32),
            pltpu.VMEM((NP,), jnp.float32),
        ],
    )
    def k(dst_hbm, out_hbm, dstv, hist):
        c = lax.axis_index("c")
        s = lax.axis_index("s")
        wid = s * NC + c

        @pl.loop(0, NP, step=16)
        def _zero(i):
            hist[pl.ds(i, 16)] = jnp.zeros((16,), jnp.float32)

        pltpu.sync_copy(dst_hbm.at[wid], dstv)

        @pl.loop(0, CPT)
        def _chunk(j):
            @pl.loop(0, CHUNK, step=16)
            def _vec(q):
                idx = dstv[j, pl.ds(q, 16)]
                plsc.addupdate_scatter(hist, [idx], jnp.ones((16,), jnp.float32))

        pltpu.sync_copy(hist, out_hbm.at[wid])

    return k(dst3)


@jax.jit
def _sc_aggregate(hp, src3, dst3):
    @functools.partial(
        pl.kernel,
        out_type=jax.ShapeDtypeStruct((NC, NP, F), jnp.float32),
        mesh=_mesh,
        scratch_types=[
            pltpu.VMEM((CPT, CHUNK), jnp.int32),
            pltpu.VMEM((8, CHUNK), jnp.int32),
            pltpu.VMEM((CHUNK, F), jnp.float32),
            pltpu.VMEM((CHUNK, F), jnp.float32),
            pltpu.VMEM_SHARED((NP, F), jnp.float32),
            pltpu.SemaphoreType.DMA,
            pltpu.SemaphoreType.DMA,
        ],
    )
    def k(hp_hbm, src_hbm, dst_hbm, out_hbm, srcv, dstv, bufa, bufb, acc,
          gs, ss):
        c = lax.axis_index("c")
        s = lax.axis_index("s")
        wid = s * NC + c
        base = s * RPS

        @pl.loop(0, CHUNK)
        def _zrow(i):
            @pl.loop(0, F, step=16)
            def _zcol(j):
                bufa[i, pl.ds(j, 16)] = jnp.zeros((16,), jnp.float32)

        @pl.loop(0, RPS, step=CHUNK)
        def _zacc(r):
            pltpu.sync_copy(bufa, acc.at[pl.ds(base + r, CHUNK)])

        plsc.subcore_barrier()

        pltpu.sync_copy(src_hbm.at[wid], srcv)
        for p in range(CPT // 8):
            pltpu.sync_copy(dst_hbm.at[wid, pl.ds(p * 8, 8)], dstv)

            @pl.loop(0, 8, step=2)
            def _edge(j):
                jj = p * 8 + j
                d0 = pltpu.async_copy(hp_hbm.at[srcv.at[jj]], bufa, gs)
                d1 = pltpu.async_copy(hp_hbm.at[srcv.at[jj + 1]], bufb, gs)
                d0.wait()
                s0 = pltpu.async_copy(bufa, acc.at[dstv.at[j]], ss, add=True)
                d1.wait()
                s1 = pltpu.async_copy(bufb, acc.at[dstv.at[j + 1]], ss,
                                      add=True)
                s0.wait()
                s1.wait()

        plsc.subcore_barrier()
        pltpu.sync_copy(acc.at[pl.ds(base, RPS)], out_hbm.at[c, pl.ds(base, RPS)])

    return k(hp, src3, dst3)


_R = 2048


def _dot(a, b):
    return jax.lax.dot_general(a, b, (((1,), (0,)), ((), ())),
                               preferred_element_type=jnp.float32)


@jax.jit
def _tc_mm(xp, W1):
    def body(x_ref, w_ref, o_ref):
        o_ref[...] = _dot(x_ref[...], w_ref[...])

    return pl.pallas_call(
        body,
        grid=(NP // _R,),
        in_specs=[
            pl.BlockSpec((_R, F), lambda i: (i, 0)),
            pl.BlockSpec((F, F), lambda i: (0, 0)),
        ],
        out_specs=pl.BlockSpec((_R, F), lambda i: (i, 0)),
        out_shape=jax.ShapeDtypeStruct((NP, F), jnp.float32),
    )(xp, W1)


@jax.jit
def _tc_scale(g, dinv):
    def body(g_ref, d_ref, o_ref):
        o_ref[...] = d_ref[...] * g_ref[...]

    return pl.pallas_call(
        body,
        grid=(NP // _R,),
        in_specs=[
            pl.BlockSpec((_R, F), lambda i: (i, 0)),
            pl.BlockSpec((_R, 1), lambda i: (i, 0)),
        ],
        out_specs=pl.BlockSpec((_R, F), lambda i: (i, 0)),
        out_shape=jax.ShapeDtypeStruct((NP, F), jnp.float32),
    )(g, dinv)


@jax.jit
def _tc_layer(p, hp, dinv, sc, sh, Wn):
    def body(p0_ref, p1_ref, hp_ref, d_ref, sc_ref, sh_ref, w_ref, o_ref):
        z = p0_ref[0] + p1_ref[0] + hp_ref[...]
        y = jnp.maximum(d_ref[...] * z * sc_ref[...] + sh_ref[...], 0.0)
        o_ref[...] = d_ref[...] * _dot(y, w_ref[...])

    return pl.pallas_call(
        body,
        grid=(NP // _R,),
        in_specs=[
            pl.BlockSpec((1, _R, F), lambda i: (0, i, 0)),
            pl.BlockSpec((1, _R, F), lambda i: (1, i, 0)),
            pl.BlockSpec((_R, F), lambda i: (i, 0)),
            pl.BlockSpec((_R, 1), lambda i: (i, 0)),
            pl.BlockSpec((1, F), lambda i: (0, 0)),
            pl.BlockSpec((1, F), lambda i: (0, 0)),
            pl.BlockSpec((F, F), lambda i: (0, 0)),
        ],
        out_specs=pl.BlockSpec((_R, F), lambda i: (i, 0)),
        out_shape=jax.ShapeDtypeStruct((NP, F), jnp.float32),
    )(p, p, hp, dinv, sc, sh, Wn)


@jax.jit
def _tc_head(p, hp, dinv, sc, sh, M1, c1, M2, c2, M3, c3, M4, c4):
    def body(p0_ref, p1_ref, hp_ref, d_ref, sc_ref, sh_ref,
             m1, c1r, m2, c2r, m3, c3r, m4, c4r, o_ref):
        z = p0_ref[0] + p1_ref[0] + hp_ref[...]
        y = jnp.maximum(d_ref[...] * z * sc_ref[...] + sh_ref[...], 0.0)
        t = jnp.maximum(_dot(y, m1[...]) + c1r[...], 0.0)
        t = jnp.maximum(_dot(t, m2[...]) + c2r[...], 0.0)
        t = jnp.maximum(_dot(t, m3[...]) + c3r[...], 0.0)
        o_ref[...] = _dot(t, m4[...]) + c4r[...]

    return pl.pallas_call(
        body,
        grid=(NP // _R,),
        in_specs=[
            pl.BlockSpec((1, _R, F), lambda i: (0, i, 0)),
            pl.BlockSpec((1, _R, F), lambda i: (1, i, 0)),
            pl.BlockSpec((_R, F), lambda i: (i, 0)),
            pl.BlockSpec((_R, 1), lambda i: (i, 0)),
            pl.BlockSpec((1, F), lambda i: (0, 0)),
            pl.BlockSpec((1, F), lambda i: (0, 0)),
            pl.BlockSpec((F, 128), lambda i: (0, 0)),
            pl.BlockSpec((1, 128), lambda i: (0, 0)),
            pl.BlockSpec((128, 64), lambda i: (0, 0)),
            pl.BlockSpec((1, 64), lambda i: (0, 0)),
            pl.BlockSpec((64, 32), lambda i: (0, 0)),
            pl.BlockSpec((1, 32), lambda i: (0, 0)),
            pl.BlockSpec((32, 16), lambda i: (0, 0)),
            pl.BlockSpec((1, 16), lambda i: (0, 0)),
        ],
        out_specs=pl.BlockSpec((_R, 16), lambda i: (i, 0)),
        out_shape=jax.ShapeDtypeStruct((N, 16), jnp.float32),
    )(p, p, hp, dinv, sc, sh, M1, c1, M2, c2, M3, c3, M4, c4)


def kernel(x, W1, b1, g1, be1, rm1, rv1, W2, b2, g2, be2, rm2, rv2,
           W3, b3, g3, be3, rm3, rv3, lw1, lb1, lw2, lb2, lw3, lb3,
           lw4, lb4, edge_index):
    ei = edge_index.astype(jnp.int32)
    pad = N + jnp.arange(EP - E, dtype=jnp.int32) % (NP - N)
    src3 = jnp.concatenate([ei[0], pad]).reshape(NT, CPT, CHUNK)
    dst3 = jnp.concatenate([ei[1], pad]).reshape(NT, CPT, CHUNK)
    xp = jnp.pad(x, ((0, NP - N), (0, 0)))

    xw1 = _tc_mm(xp, W1)
    hist = _sc_degree(dst3)
    deg = jnp.sum(hist, axis=0) + 1.0
    dinv = jnp.where(jnp.arange(NP) < N, lax.rsqrt(deg), 0.0).reshape(NP, 1)

    def bn_consts(b, g, be, rm, rv):
        s = g * lax.rsqrt(rv + 1e-5)
        return s.reshape(1, F), ((b - rm) * s + be).reshape(1, F)

    s1, t1 = bn_consts(b1, g1, be1, rm1, rv1)
    s2, t2 = bn_consts(b2, g2, be2, rm2, rv2)
    s3, t3 = bn_consts(b3, g3, be3, rm3, rv3)

    hp = _tc_scale(xw1, dinv)
    p = _sc_aggregate(hp, src3, dst3)
    hp = _tc_layer(p, hp, dinv, s1, t1, W2)
    p = _sc_aggregate(hp, src3, dst3)
    hp = _tc_layer(p, hp, dinv, s2, t2, W3)
    p = _sc_aggregate(hp, src3, dst3)
    out = _tc_head(p, hp, dinv, s3, t3,
                   lw1.T, lb1.reshape(1, -1), lw2.T, lb2.reshape(1, -1),
                   lw3.T, lb3.reshape(1, -1), lw4.T, lb4.reshape(1, -1))
    return out
